# baseline (speedup 1.0000x reference)
import math

import jax
import jax.numpy as jnp
from jax import lax
from jax.experimental import pallas as pl
from jax.experimental.pallas import tpu as pltpu
from jax.experimental.pallas import tpu_sc as plsc

VOCAB = 100000
HIDDEN = 128
B = 1024
L = 200
NC = 2
NS = 16
NW = NC * NS
B_PER_W = B // NW
SCALE = 1.0 / math.sqrt(HIDDEN)
NVH = HIDDEN // 16


def _emb_body(x_hbm, table_hbm, pos_hbm, out_hbm,
              pos_v, x_v, buf0, buf1, buf2,
              gsem0, gsem1, gsem2, osem0, osem1, osem2):
    wid = lax.axis_index("s") * NC + lax.axis_index("c")
    base = wid * B_PER_W

    pltpu.sync_copy(x_hbm.at[pl.ds(base * L, B_PER_W * L)], x_v)
    pltpu.sync_copy(pos_hbm.at[pl.ds(0, L)], pos_v)

    def scale_body(t, _):
        for h in range(NVH):
            sl = pl.ds(h * 16, 16)
            pos_v[t, sl] = pos_v[t, sl] * SCALE
        return _

    lax.fori_loop(0, L, scale_body, 0)

    def add_body_for(buf):
        def add_body(t, _):
            for h in range(NVH):
                sl = pl.ds(h * 16, 16)
                buf[t, sl] = buf[t, sl] + pos_v[t, sl]
            return _
        return add_body

    slots = ((buf0, gsem0, osem0), (buf1, gsem1, osem1), (buf2, gsem2, osem2))
    NSL = len(slots)
    out_cp = [None] * NSL
    gather_cp = [None] * NSL

    def start_gather(j):
        k = j % NSL
        gather_cp[k] = pltpu.async_copy(
            table_hbm.at[x_v.at[pl.ds(j * L, L)]], slots[k][0], slots[k][1])

    start_gather(0)
    start_gather(1)

    for j in range(B_PER_W):
        k = j % NSL
        buf, _, osem = slots[k]
        gather_cp[k].wait()
        lax.fori_loop(0, L, add_body_for(buf), 0, unroll=4)
        out_cp[k] = pltpu.async_copy(
            buf, out_hbm.at[pl.ds((base + j) * L, L)], osem)
        if j + 2 < B_PER_W:
            nk = (j + 2) % NSL
            if out_cp[nk] is not None:
                out_cp[nk].wait()
            start_gather(j + 2)

    for k in range(NSL):
        if out_cp[k] is not None:
            out_cp[k].wait()


@jax.jit
def _emb(x_flat, table, pos_weight):
    mesh = plsc.VectorSubcoreMesh(core_axis_name="c", subcore_axis_name="s",
                                  num_cores=NC, num_subcores=NS)
    return pl.kernel(
        _emb_body,
        out_type=jax.ShapeDtypeStruct((B * L, HIDDEN), jnp.float32),
        mesh=mesh,
        scratch_types=[
            pltpu.VMEM((L, HIDDEN), jnp.float32),
            pltpu.VMEM((B_PER_W * L,), jnp.int32),
            pltpu.VMEM((L, HIDDEN), jnp.float32),
            pltpu.VMEM((L, HIDDEN), jnp.float32),
            pltpu.VMEM((L, HIDDEN), jnp.float32),
            pltpu.SemaphoreType.DMA,
            pltpu.SemaphoreType.DMA,
            pltpu.SemaphoreType.DMA,
            pltpu.SemaphoreType.DMA,
            pltpu.SemaphoreType.DMA,
            pltpu.SemaphoreType.DMA,
        ],
    )(x_flat, table, pos_weight)


def kernel(X, table, pos_weight):
    x_flat = X.reshape(B * L).astype(jnp.int32)
    out = _emb(x_flat, table, pos_weight)
    return out.reshape(B, L, HIDDEN)


# speedup vs baseline: 2.6395x; 2.6395x over previous
import math

import jax
import jax.numpy as jnp
from jax import lax
from jax.experimental import pallas as pl
from jax.experimental.pallas import tpu as pltpu
from jax.experimental.pallas import tpu_sc as plsc

VOCAB = 100000
HIDDEN = 128
B = 1024
L = 200
NC = 2
NS = 16
NW = NC * NS
B_PER_W = B // NW
SCALE = 1.0 / math.sqrt(HIDDEN)
NVH = HIDDEN // 16


def _emb_body(x_hbm, table_hbm, pos_hbm, out_hbm,
              pos_v, x_v, buf0, buf1, buf2,
              gsem0, gsem1, gsem2, osem0, osem1, osem2):
    wid = lax.axis_index("s") * NC + lax.axis_index("c")
    base = wid * B_PER_W

    pltpu.sync_copy(x_hbm.at[pl.ds(base * L, B_PER_W * L)], x_v)
    pltpu.sync_copy(pos_hbm.at[pl.ds(0, L)], pos_v)

    def scale_body(t, _):
        for h in range(NVH):
            sl = pl.ds(h * 16, 16)
            pos_v[t, sl] = pos_v[t, sl] * SCALE
        return _

    lax.fori_loop(0, L, scale_body, 0)

    def add_body_for(buf):
        def add_body(t, _):
            for h in range(NVH):
                sl = pl.ds(h * 16, 16)
                buf[t, sl] = buf[t, sl] + pos_v[t, sl]
            return _
        return add_body

    slots = ((buf0, gsem0, osem0), (buf1, gsem1, osem1), (buf2, gsem2, osem2))
    NSL = len(slots)
    out_cp = [None] * NSL
    gather_cp = [None] * NSL

    def start_gather(j):
        k = j % NSL
        gather_cp[k] = pltpu.async_copy(
            table_hbm.at[x_v.at[pl.ds(j * L, L)]], slots[k][0], slots[k][1])

    start_gather(0)
    start_gather(1)

    for j in range(B_PER_W):
        k = j % NSL
        buf, _, osem = slots[k]
        gather_cp[k].wait()
        lax.fori_loop(0, L, add_body_for(buf), 0)
        out_cp[k] = pltpu.async_copy(
            buf, out_hbm.at[pl.ds((base + j) * L, L)], osem)
        if j + 2 < B_PER_W:
            nk = (j + 2) % NSL
            if out_cp[nk] is not None:
                out_cp[nk].wait()
            start_gather(j + 2)

    for k in range(NSL):
        if out_cp[k] is not None:
            out_cp[k].wait()


@jax.jit
def _emb(x_flat, table, pos_weight):
    mesh = plsc.VectorSubcoreMesh(core_axis_name="c", subcore_axis_name="s",
                                  num_cores=NC, num_subcores=NS)
    return pl.kernel(
        _emb_body,
        out_type=jax.ShapeDtypeStruct((B * L, HIDDEN), jnp.float32),
        mesh=mesh,
        scratch_types=[
            pltpu.VMEM((L, HIDDEN), jnp.float32),
            pltpu.VMEM((B_PER_W * L,), jnp.int32),
            pltpu.VMEM((L, HIDDEN), jnp.float32),
            pltpu.VMEM((L, HIDDEN), jnp.float32),
            pltpu.VMEM((L, HIDDEN), jnp.float32),
            pltpu.SemaphoreType.DMA,
            pltpu.SemaphoreType.DMA,
            pltpu.SemaphoreType.DMA,
            pltpu.SemaphoreType.DMA,
            pltpu.SemaphoreType.DMA,
            pltpu.SemaphoreType.DMA,
        ],
    )(x_flat, table, pos_weight)


def kernel(X, table, pos_weight):
    x_flat = X.reshape(B * L).astype(jnp.int32)
    out = _emb(x_flat, table, pos_weight)
    return out.reshape(B, L, HIDDEN)
